# baseline (speedup 1.0000x reference)
import jax
import jax.numpy as jnp
from jax import lax
from jax.experimental import pallas as pl
from jax.experimental.pallas import tpu as pltpu
from jax.experimental.pallas import tpu_sc as plsc

_B = 16384
_C = 1000
_NC = 2
_NS = 16
_NW = _NC * _NS
_ROWS_W = _B // _NW
_CHUNK = 32
_NCHUNK = _ROWS_W // _CHUNK
_CHUNK_ELEMS = _CHUNK * _C
_LANES = 16


def _onehot_body(x_hbm, out_hbm, idx_v, buf0, buf1, sem0, sem1):
    wid = lax.axis_index("s") * _NC + lax.axis_index("c")
    base_row = wid * _ROWS_W

    pltpu.sync_copy(x_hbm.at[pl.ds(base_row, _ROWS_W)], idx_v)

    zeros16 = jnp.zeros((_LANES,), jnp.float32)
    ones16 = jnp.ones((_LANES,), jnp.float32)
    lane = lax.iota(jnp.int32, _LANES)

    def _zero(i, carry):
        for j in range(8):
            off = (i * 8 + j) * _LANES
            buf0[pl.ds(off, _LANES)] = zeros16
            buf1[pl.ds(off, _LANES)] = zeros16
        return carry

    lax.fori_loop(0, _CHUNK_ELEMS // (_LANES * 8), _zero, 0)

    def _positions(c, j):
        iv = idx_v[pl.ds(c * _CHUNK + j * _LANES, _LANES)]
        lrow = lane + (j * _LANES)
        return lrow * _C + iv

    bufs = (buf0, buf1)
    sems = (sem0, sem1)
    copies = [None, None]
    for c in range(_NCHUNK):
        b = c % 2
        buf = bufs[b]
        if c >= 2:
            copies[b].wait()
            for j in range(_CHUNK // _LANES):
                plsc.store_scatter(buf, [_positions(c - 2, j)], zeros16)
        for j in range(_CHUNK // _LANES):
            plsc.store_scatter(buf, [_positions(c, j)], ones16)
        dst = out_hbm.at[pl.ds((base_row + c * _CHUNK) * _C, _CHUNK_ELEMS)]
        copies[b] = pltpu.async_copy(buf, dst, sems[b])
    copies[0].wait()
    copies[1].wait()


def kernel(x):
    x_flat = x.reshape(_B)
    mesh = plsc.VectorSubcoreMesh(core_axis_name="c", subcore_axis_name="s")
    f = pl.kernel(
        _onehot_body,
        out_type=jax.ShapeDtypeStruct((_B * _C,), jnp.float32),
        mesh=mesh,
        compiler_params=pltpu.CompilerParams(needs_layout_passes=False),
        scratch_types=[
            pltpu.VMEM((_ROWS_W,), jnp.int32),
            pltpu.VMEM((_CHUNK_ELEMS,), jnp.float32),
            pltpu.VMEM((_CHUNK_ELEMS,), jnp.float32),
            pltpu.SemaphoreType.DMA,
            pltpu.SemaphoreType.DMA,
        ],
    )
    out = f(x_flat)
    return out.reshape(_B, _C)


# speedup vs baseline: 1.5213x; 1.5213x over previous
import jax
import jax.numpy as jnp
from jax import lax
from jax.experimental import pallas as pl
from jax.experimental.pallas import tpu as pltpu
from jax.experimental.pallas import tpu_sc as plsc

_B = 16384
_C = 1000
_NC = 2
_NS = 16
_NW = _NC * _NS
_ROWS_W = _B // _NW
_CHUNK = 32
_NCHUNK = _ROWS_W // _CHUNK
_LANES = 16


def _onehot_body(x_hbm, out_hbm, idx_v, buf0, buf1, sem0, sem1):
    wid = lax.axis_index("s") * _NC + lax.axis_index("c")
    base_row = wid * _ROWS_W

    pltpu.sync_copy(x_hbm.at[pl.ds(base_row, _ROWS_W)], idx_v)

    zeros16 = jnp.zeros((_LANES,), jnp.float32)
    ones16 = jnp.ones((_LANES,), jnp.float32)
    lane = lax.iota(jnp.int32, _LANES)
    zlane = jnp.zeros((_LANES,), jnp.int32)
    tail_mask = lane < (_C % _LANES)

    def _zero(r, carry):
        for j in range(_C // _LANES):
            buf0[r, pl.ds(j * _LANES, _LANES)] = zeros16
            buf1[r, pl.ds(j * _LANES, _LANES)] = zeros16
        rfull = jnp.full((_LANES,), r, jnp.int32)
        tail = (_C // _LANES) * _LANES + lane
        plsc.store_scatter(buf0, [rfull, tail], zeros16, mask=tail_mask)
        plsc.store_scatter(buf1, [rfull, tail], zeros16, mask=tail_mask)
        return carry

    lax.fori_loop(0, _CHUNK, _zero, 0)

    def _scatter(buf, c, val):
        for j in range(_CHUNK // _LANES):
            row = c * _CHUNK + j * _LANES + lane
            iv = plsc.load_gather(idx_v, [row, zlane])
            lrow = j * _LANES + lane
            plsc.store_scatter(buf, [lrow, iv], val)

    bufs = (buf0, buf1)
    sems = (sem0, sem1)
    copies = [None, None]
    for c in range(_NCHUNK):
        b = c % 2
        buf = bufs[b]
        if c >= 2:
            copies[b].wait()
            _scatter(buf, c - 2, zeros16)
        _scatter(buf, c, ones16)
        dst = out_hbm.at[pl.ds(base_row + c * _CHUNK, _CHUNK)]
        copies[b] = pltpu.async_copy(buf, dst, sems[b])
    copies[0].wait()
    copies[1].wait()


def kernel(x):
    mesh = plsc.VectorSubcoreMesh(core_axis_name="c", subcore_axis_name="s")
    f = pl.kernel(
        _onehot_body,
        out_type=jax.ShapeDtypeStruct((_B, _C), jnp.float32),
        mesh=mesh,
        compiler_params=pltpu.CompilerParams(needs_layout_passes=False),
        scratch_types=[
            pltpu.VMEM((_ROWS_W, 1), jnp.int32),
            pltpu.VMEM((_CHUNK, _C), jnp.float32),
            pltpu.VMEM((_CHUNK, _C), jnp.float32),
            pltpu.SemaphoreType.DMA,
            pltpu.SemaphoreType.DMA,
        ],
    )
    return f(x)


# speedup vs baseline: 3.5756x; 2.3503x over previous
import jax
import jax.numpy as jnp
from jax import lax
from jax.experimental import pallas as pl
from jax.experimental.pallas import tpu as pltpu
from jax.experimental.pallas import tpu_sc as plsc

_B = 16384
_C = 1000
_NC = 2
_NS = 16
_NW = _NC * _NS
_COLS_W = _B // _NW
_CCHUNK = 128
_NCC = _COLS_W // _CCHUNK
_RCHUNK = 200
_NRC = _C // _RCHUNK
_LANES = 16


def _onehot_body(x_hbm, out_hbm, idx_v, buf0, buf1, sem0, sem1):
    wid = lax.axis_index("s") * _NC + lax.axis_index("c")
    base_col = wid * _COLS_W

    pltpu.sync_copy(x_hbm.at[pl.ds(base_col, _COLS_W)], idx_v)

    zeros16 = jnp.zeros((_LANES,), jnp.float32)
    ones16 = jnp.ones((_LANES,), jnp.float32)
    lane = lax.iota(jnp.int32, _LANES)

    def _zero(r, carry):
        for j in range(_CCHUNK // _LANES):
            buf0[r, pl.ds(j * _LANES, _LANES)] = zeros16
            buf1[r, pl.ds(j * _LANES, _LANES)] = zeros16
        return carry

    lax.fori_loop(0, _RCHUNK, _zero, 0)

    def _scatter(buf, cc, rc, val):
        r0 = rc * _RCHUNK
        for j in range(_CCHUNK // _LANES):
            iv = idx_v[pl.ds(cc * _CCHUNK + j * _LANES, _LANES)]
            m = (iv >= r0) & (iv < r0 + _RCHUNK)
            plsc.store_scatter(buf, [iv - r0, j * _LANES + lane], val, mask=m)

    bufs = (buf0, buf1)
    sems = (sem0, sem1)
    copies = [None, None]
    steps = [(cc, rc) for cc in range(_NCC) for rc in range(_NRC)]
    for s, (cc, rc) in enumerate(steps):
        b = s % 2
        buf = bufs[b]
        if s >= 2:
            copies[b].wait()
            pcc, prc = steps[s - 2]
            _scatter(buf, pcc, prc, zeros16)
        _scatter(buf, cc, rc, ones16)
        dst = out_hbm.at[
            pl.ds(rc * _RCHUNK, _RCHUNK),
            pl.ds(base_col + cc * _CCHUNK, _CCHUNK),
        ]
        copies[b] = pltpu.async_copy(buf, dst, sems[b])
    copies[0].wait()
    copies[1].wait()


def kernel(x):
    mesh = plsc.VectorSubcoreMesh(core_axis_name="c", subcore_axis_name="s")
    f = pl.kernel(
        _onehot_body,
        out_type=jax.ShapeDtypeStruct((_C, _B), jnp.float32),
        mesh=mesh,
        compiler_params=pltpu.CompilerParams(needs_layout_passes=False),
        scratch_types=[
            pltpu.VMEM((_COLS_W,), jnp.int32),
            pltpu.VMEM((_RCHUNK, _CCHUNK), jnp.float32),
            pltpu.VMEM((_RCHUNK, _CCHUNK), jnp.float32),
            pltpu.SemaphoreType.DMA,
            pltpu.SemaphoreType.DMA,
        ],
    )
    out_t = f(x.reshape(_B))
    return jnp.transpose(out_t)


# speedup vs baseline: 3.6260x; 1.0141x over previous
import jax
import jax.numpy as jnp
from jax import lax
from jax.experimental import pallas as pl
from jax.experimental.pallas import tpu as pltpu
from jax.experimental.pallas import tpu_sc as plsc

_B = 16384
_C = 1000
_NC = 2
_NS = 16
_NW = _NC * _NS
_COLS_W = _B // _NW
_CCHUNK = 128
_NCC = _COLS_W // _CCHUNK
_RCHUNK = 200
_NRC = _C // _RCHUNK
_LANES = 16


def _onehot_body(x_hbm, out_hbm, idx_v, buf0, buf1, sem0, sem1):
    wid = lax.axis_index("s") * _NC + lax.axis_index("c")
    base_col = wid * _COLS_W

    pltpu.sync_copy(x_hbm.at[pl.ds(base_col, _COLS_W)], idx_v)

    zeros16 = jnp.zeros((_LANES,), jnp.float32)
    ones16 = jnp.ones((_LANES,), jnp.float32)
    lane = lax.iota(jnp.int32, _LANES)

    def _zero_slab(buf):
        def _zero(r, carry):
            for j in range(_CCHUNK // _LANES):
                buf[r, pl.ds(j * _LANES, _LANES)] = zeros16
            return carry

        lax.fori_loop(0, _RCHUNK, _zero, 0)

    def _scatter(buf, cc, rc, val):
        r0 = rc * _RCHUNK
        for j in range(_CCHUNK // _LANES):
            iv = idx_v[pl.ds(cc * _CCHUNK + j * _LANES, _LANES)]
            m = (iv >= r0) & (iv < r0 + _RCHUNK)
            plsc.store_scatter(buf, [iv - r0, j * _LANES + lane], val, mask=m)

    bufs = (buf0, buf1)
    sems = (sem0, sem1)
    copies = [None, None]
    steps = [(cc, rc) for cc in range(_NCC) for rc in range(_NRC)]
    for s, (cc, rc) in enumerate(steps):
        b = s % 2
        buf = bufs[b]
        if s < 2:
            _zero_slab(buf)
        if s >= 2:
            copies[b].wait()
            pcc, prc = steps[s - 2]
            _scatter(buf, pcc, prc, zeros16)
        _scatter(buf, cc, rc, ones16)
        dst = out_hbm.at[
            pl.ds(rc * _RCHUNK, _RCHUNK),
            pl.ds(base_col + cc * _CCHUNK, _CCHUNK),
        ]
        copies[b] = pltpu.async_copy(buf, dst, sems[b])
    copies[0].wait()
    copies[1].wait()


def kernel(x):
    mesh = plsc.VectorSubcoreMesh(core_axis_name="c", subcore_axis_name="s")
    f = pl.kernel(
        _onehot_body,
        out_type=jax.ShapeDtypeStruct((_C, _B), jnp.float32),
        mesh=mesh,
        compiler_params=pltpu.CompilerParams(
            needs_layout_passes=False,
            skip_device_barrier=True,
            disable_semaphore_checks=True,
        ),
        scratch_types=[
            pltpu.VMEM((_COLS_W,), jnp.int32),
            pltpu.VMEM((_RCHUNK, _CCHUNK), jnp.float32),
            pltpu.VMEM((_RCHUNK, _CCHUNK), jnp.float32),
            pltpu.SemaphoreType.DMA,
            pltpu.SemaphoreType.DMA,
        ],
    )
    out_t = f(x.reshape(_B))
    return jnp.transpose(out_t)


# speedup vs baseline: 3.6801x; 1.0149x over previous
import jax
import jax.numpy as jnp
from jax import lax
from jax.experimental import pallas as pl
from jax.experimental.pallas import tpu as pltpu
from jax.experimental.pallas import tpu_sc as plsc

_B = 16384
_C = 1000
_NC = 2
_NS = 16
_NW = _NC * _NS
_COLS_W = _B // _NW
_CCHUNK = 128
_NCC = _COLS_W // _CCHUNK
_RCHUNK = 200
_NRC = _C // _RCHUNK
_LANES = 16


def _onehot_body(x_hbm, out_hbm, idx_v, buf0, buf1, buf2, sem0, sem1, sem2, isem):
    wid = lax.axis_index("s") * _NC + lax.axis_index("c")
    base_col = wid * _COLS_W

    idx_copy = pltpu.async_copy(x_hbm.at[pl.ds(base_col, _COLS_W)], idx_v, isem)

    zeros16 = jnp.zeros((_LANES,), jnp.float32)
    ones16 = jnp.ones((_LANES,), jnp.float32)
    lane = lax.iota(jnp.int32, _LANES)

    def _zero_slab(buf):
        def _zero(r, carry):
            for j in range(_CCHUNK // _LANES):
                buf[r, pl.ds(j * _LANES, _LANES)] = zeros16
            return carry

        lax.fori_loop(0, _RCHUNK, _zero, 0)

    def _scatter(buf, cc, rc, val):
        r0 = rc * _RCHUNK
        for j in range(_CCHUNK // _LANES):
            iv = idx_v[pl.ds(cc * _CCHUNK + j * _LANES, _LANES)]
            m = (iv >= r0) & (iv < r0 + _RCHUNK)
            plsc.store_scatter(buf, [iv - r0, j * _LANES + lane], val, mask=m)

    bufs = (buf0, buf1, buf2)
    sems = (sem0, sem1, sem2)
    nbuf = len(bufs)
    copies = [None] * nbuf
    steps = [(cc, rc) for cc in range(_NCC) for rc in range(_NRC)]
    for s, (cc, rc) in enumerate(steps):
        b = s % nbuf
        buf = bufs[b]
        if s < nbuf:
            _zero_slab(buf)
        if s == 0:
            idx_copy.wait()
        if s >= nbuf:
            copies[b].wait()
            pcc, prc = steps[s - nbuf]
            _scatter(buf, pcc, prc, zeros16)
        _scatter(buf, cc, rc, ones16)
        dst = out_hbm.at[
            pl.ds(rc * _RCHUNK, _RCHUNK),
            pl.ds(base_col + cc * _CCHUNK, _CCHUNK),
        ]
        copies[b] = pltpu.async_copy(buf, dst, sems[b])
    for cp in copies:
        cp.wait()


def kernel(x):
    mesh = plsc.VectorSubcoreMesh(core_axis_name="c", subcore_axis_name="s")
    f = pl.kernel(
        _onehot_body,
        out_type=jax.ShapeDtypeStruct((_C, _B), jnp.float32),
        mesh=mesh,
        compiler_params=pltpu.CompilerParams(
            needs_layout_passes=False,
            skip_device_barrier=True,
            disable_semaphore_checks=True,
        ),
        scratch_types=[
            pltpu.VMEM((_COLS_W,), jnp.int32),
            pltpu.VMEM((_RCHUNK, _CCHUNK), jnp.float32),
            pltpu.VMEM((_RCHUNK, _CCHUNK), jnp.float32),
            pltpu.VMEM((_RCHUNK, _CCHUNK), jnp.float32),
            pltpu.SemaphoreType.DMA,
            pltpu.SemaphoreType.DMA,
            pltpu.SemaphoreType.DMA,
            pltpu.SemaphoreType.DMA,
        ],
    )
    out_t = f(x.reshape(_B))
    return jnp.transpose(out_t)


# speedup vs baseline: 3.9067x; 1.0616x over previous
import jax
import jax.numpy as jnp
from jax import lax
from jax.experimental import pallas as pl
from jax.experimental.pallas import tpu as pltpu
from jax.experimental.pallas import tpu_sc as plsc

_B = 16384
_C = 1000
_NC = 2
_NS = 16
_NW = _NC * _NS
_COLS_W = _B // _NW
_CCHUNK = 128
_NCC = _COLS_W // _CCHUNK
_RCHUNK = 200
_NRC = _C // _RCHUNK
_LANES = 16
_NBUF = 4
_NSTEP = _NCC * _NRC
_NOUTER = _NSTEP // _NBUF


def _onehot_body(x_hbm, out_hbm, idx_v, b0, b1, b2, b3, s0, s1, s2, s3, isem):
    wid = lax.axis_index("s") * _NC + lax.axis_index("c")
    base_col = wid * _COLS_W

    idx_copy = pltpu.async_copy(x_hbm.at[pl.ds(base_col, _COLS_W)], idx_v, isem)

    zeros16 = jnp.zeros((_LANES,), jnp.float32)
    ones16 = jnp.ones((_LANES,), jnp.float32)
    lane = lax.iota(jnp.int32, _LANES)
    bufs = (b0, b1, b2, b3)
    sems = (s0, s1, s2, s3)

    def _zero_slab(buf):
        def _zero(r, carry):
            for j in range(_CCHUNK // _LANES):
                buf[r, pl.ds(j * _LANES, _LANES)] = zeros16
            return carry

        lax.fori_loop(0, _RCHUNK, _zero, 0)

    def _dst(s):
        cc = s // _NRC
        rc = s % _NRC
        return out_hbm.at[
            pl.ds(rc * _RCHUNK, _RCHUNK),
            pl.ds(base_col + cc * _CCHUNK, _CCHUNK),
        ]

    def _scatter(buf, s, val):
        cc = s // _NRC
        r0 = (s % _NRC) * _RCHUNK
        for j in range(_CCHUNK // _LANES):
            iv = idx_v[pl.ds(cc * _CCHUNK + j * _LANES, _LANES)]
            m = (iv >= r0) & (iv < r0 + _RCHUNK)
            plsc.store_scatter(buf, [iv - r0, j * _LANES + lane], val, mask=m)

    def _outer(g, carry):
        for k in range(_NBUF):
            s = g * _NBUF + k
            buf = bufs[k]
            sem = sems[k]

            @pl.when(g == 0)
            def _():
                _zero_slab(buf)

            if k == 0:

                @pl.when(g == 0)
                def _():
                    idx_copy.wait()

            @pl.when(g > 0)
            def _():
                pltpu.make_async_copy(buf, _dst(s - _NBUF), sem).wait()
                _scatter(buf, s - _NBUF, zeros16)

            _scatter(buf, s, ones16)
            pltpu.async_copy(buf, _dst(s), sem)
        return carry

    lax.fori_loop(0, _NOUTER, _outer, 0)

    for k in range(_NBUF):
        s = (_NOUTER - 1) * _NBUF + k
        pltpu.make_async_copy(bufs[k], _dst(s), sems[k]).wait()


def kernel(x):
    mesh = plsc.VectorSubcoreMesh(core_axis_name="c", subcore_axis_name="s")
    f = pl.kernel(
        _onehot_body,
        out_type=jax.ShapeDtypeStruct((_C, _B), jnp.float32),
        mesh=mesh,
        compiler_params=pltpu.CompilerParams(
            needs_layout_passes=False,
            skip_device_barrier=True,
            disable_semaphore_checks=True,
        ),
        scratch_types=[
            pltpu.VMEM((_COLS_W,), jnp.int32),
            pltpu.VMEM((_RCHUNK, _CCHUNK), jnp.float32),
            pltpu.VMEM((_RCHUNK, _CCHUNK), jnp.float32),
            pltpu.VMEM((_RCHUNK, _CCHUNK), jnp.float32),
            pltpu.VMEM((_RCHUNK, _CCHUNK), jnp.float32),
            pltpu.SemaphoreType.DMA,
            pltpu.SemaphoreType.DMA,
            pltpu.SemaphoreType.DMA,
            pltpu.SemaphoreType.DMA,
            pltpu.SemaphoreType.DMA,
        ],
    )
    out_t = f(x.reshape(_B))
    return jnp.transpose(out_t)
